# baseline (speedup 1.0000x reference)
import functools

import jax
import jax.numpy as jnp
from jax import lax
from jax.experimental import pallas as pl
from jax.experimental.pallas import tpu as pltpu
from jax.experimental.pallas import tpu_sc as plsc

_N, _F, _H, _C, _E = 10000, 128, 16, 16, 320000
_NPAD = 10240
_K = 128
_NSC, _NTILE = 2, 16
_NW = _NSC * _NTILE
_NCH = 80
_EPW = _NCH * _K
_EPAD = _NW * _EPW
_RPT = _NPAD // _NTILE
_NBUF = 8


def _tc_pre(x, wlt, wrt, bl):
    def body(x_ref, wl_ref, wr_ref, bl_ref, y_ref, z_ref):
        xv = x_ref[...]
        y_ref[...] = jnp.dot(xv, wl_ref[...], preferred_element_type=jnp.float32)
        z_ref[...] = (
            jnp.dot(xv, wr_ref[...], preferred_element_type=jnp.float32)
            + bl_ref[...]
        )

    return pl.pallas_call(
        body,
        out_shape=[
            jax.ShapeDtypeStruct((_N, _H), jnp.float32),
            jax.ShapeDtypeStruct((_N, _H), jnp.float32),
        ],
    )(x, wlt, wrt, bl)


def _sc_agg(y, src, dst2d):
    mesh = plsc.VectorSubcoreMesh(core_axis_name="c", subcore_axis_name="s")

    @functools.partial(
        pl.kernel,
        out_type=[
            jax.ShapeDtypeStruct((_NSC, _NPAD, _H), jnp.float32),
            jax.ShapeDtypeStruct((_NSC, _NPAD), jnp.float32),
        ],
        mesh=mesh,
        compiler_params=pltpu.CompilerParams(use_tc_tiling_on_sc=False),
        scratch_types=[
            pltpu.VMEM_SHARED((_NPAD, _H), jnp.float32),
            pltpu.VMEM_SHARED((_NPAD,), jnp.float32),
            pltpu.VMEM((_EPW,), jnp.int32),
            pltpu.VMEM((_NCH, _K), jnp.int32),
            pltpu.VMEM((_NBUF, _K, _H), jnp.float32),
            pltpu.VMEM((_K,), jnp.float32),
            pltpu.VMEM((_K,), jnp.float32),
            pltpu.SemaphoreType.DMA,
            pltpu.SemaphoreType.DMA((_NBUF,)),
            pltpu.SemaphoreType.DMA((_NBUF,)),
            pltpu.SemaphoreType.DMA,
        ],
    )
    def k(y_hbm, src_hbm, dst_hbm, acc_out, deg_out,
          acc_sh, deg_sh, srcv, dstv, rows, onev, zerov,
          semi, semg, sems, semd):
        cid = lax.axis_index("c")
        sid = lax.axis_index("s")
        wid = cid * _NTILE + sid

        cp_s = pltpu.async_copy(src_hbm.at[pl.ds(wid * _EPW, _EPW)], srcv, semi)
        cp_d = pltpu.async_copy(dst_hbm.at[pl.ds(wid * _NCH, _NCH)], dstv, semi)

        def fill_rows(i, _):
            rows[0, i] = jnp.zeros((_H,), jnp.float32)
            return 0

        lax.fori_loop(0, _K, fill_rows, 0)

        def fill_vecs(i, _):
            zerov[pl.ds(i * 16, 16)] = jnp.zeros((16,), jnp.float32)
            onev[pl.ds(i * 16, 16)] = jnp.ones((16,), jnp.float32)
            return 0

        lax.fori_loop(0, _K // 16, fill_vecs, 0)

        def zero_stripe(i, _):
            off = sid * _RPT + i * _K
            pltpu.sync_copy(rows.at[0], acc_sh.at[pl.ds(off, _K)])
            pltpu.sync_copy(zerov, deg_sh.at[pl.ds(off, _K)])
            return 0

        lax.fori_loop(0, _RPT // _K, zero_stripe, 0)
        cp_s.wait()
        cp_d.wait()
        plsc.subcore_barrier()

        def gather(j, b):
            sidx = srcv.at[pl.ds(j * _K, _K)]
            pltpu.async_copy(y_hbm.at[sidx], rows.at[b], semg.at[b])

        for b in range(_NBUF):
            gather(b, b)

        half = _NBUF // 2

        def outer(i, _):
            j0 = i * _NBUF
            for b in range(_NBUF):
                j = j0 + b
                sidx = srcv.at[pl.ds(j * _K, _K)]
                pltpu.make_async_copy(
                    y_hbm.at[sidx], rows.at[b], semg.at[b]).wait()
                dvi = dstv.at[j]
                pltpu.async_copy(rows.at[b], acc_sh.at[dvi], sems.at[b],
                                 add=True)
                pltpu.async_copy(onev, deg_sh.at[dvi], semd, add=True)

                @pl.when(j >= half)
                def _():
                    pltpu.make_async_copy(
                        onev, deg_sh.at[dvi], semd).wait()

                bp = (b + half) % _NBUF

                @pl.when((j >= half) & (j + half < _NCH))
                def _():
                    pltpu.make_async_copy(
                        rows.at[bp], acc_sh.at[dvi], sems.at[bp]).wait()
                    gather(j + half, bp)

            return 0

        lax.fori_loop(0, _NCH // _NBUF, outer, 0)

        for b in range(half):
            pltpu.make_async_copy(onev, deg_sh.at[dstv.at[0]], semd).wait()
        for b in range(_NBUF):
            pltpu.make_async_copy(
                rows.at[b], acc_sh.at[dstv.at[0]], sems.at[b]).wait()
        plsc.subcore_barrier()

        off = sid * _RPT
        pltpu.sync_copy(acc_sh.at[pl.ds(off, _RPT)],
                        acc_out.at[cid, pl.ds(off, _RPT)])
        pltpu.sync_copy(deg_sh.at[pl.ds(off, _RPT)],
                        deg_out.at[cid, pl.ds(off, _RPT)])

    return k(y, src, dst2d)


def _tc_post(acc, deg3, z, wfct, bfc):
    def body(acc_ref, deg_ref, z_ref, w_ref, b_ref, out_ref):
        a = acc_ref[0] + acc_ref[1]
        d = deg_ref[0] + deg_ref[1]
        h = jnp.maximum(a[:_N] / jnp.maximum(d[:_N], 1.0) + z_ref[...], 0.0)
        out_ref[...] = (
            jnp.dot(h, w_ref[...], preferred_element_type=jnp.float32)
            + b_ref[...]
        )

    return pl.pallas_call(
        body,
        out_shape=jax.ShapeDtypeStruct((_N, _C), jnp.float32),
    )(acc, deg3, z, wfct, bfc)


def kernel(x, edge_index, W_l, b_l, W_r, W_fc, b_fc):
    npad = _EPAD - _E
    src = jnp.concatenate([edge_index[0], jnp.zeros((npad,), jnp.int32)])
    dst = jnp.concatenate(
        [edge_index[1], jnp.full((npad,), _N, jnp.int32)])
    dst2d = dst.reshape(_EPAD // _K, _K)
    y, z = _tc_pre(x, W_l.T, W_r.T, b_l.reshape(1, _H))
    acc, deg = _sc_agg(y, src, dst2d)
    return _tc_post(acc, deg.reshape(_NSC, _NPAD, 1), z, W_fc.T, b_fc.reshape(1, _C))


# speedup vs baseline: 1.3743x; 1.3743x over previous
import functools

import jax
import jax.numpy as jnp
from jax import lax
from jax.experimental import pallas as pl
from jax.experimental.pallas import tpu as pltpu
from jax.experimental.pallas import tpu_sc as plsc

_N, _F, _H, _C, _E = 10000, 128, 16, 16, 320000
_NPAD = 10240
_K = 128
_NSC, _NTILE = 2, 16
_NW = _NSC * _NTILE
_NCH = 80
_EPW = _NCH * _K
_EPAD = _NW * _EPW
_RPT = _NPAD // _NTILE
_NBUF = 8


def _tc_pre(x, wlt, wrt, bl):
    def body(x_ref, wl_ref, wr_ref, bl_ref, y_ref, z_ref):
        xv = x_ref[...]
        y_ref[...] = jnp.dot(xv, wl_ref[...], preferred_element_type=jnp.float32)
        z_ref[...] = (
            jnp.dot(xv, wr_ref[...], preferred_element_type=jnp.float32)
            + bl_ref[...]
        )

    return pl.pallas_call(
        body,
        out_shape=[
            jax.ShapeDtypeStruct((_N, _H), jnp.float32),
            jax.ShapeDtypeStruct((_N, _H), jnp.float32),
        ],
    )(x, wlt, wrt, bl)


def _sc_agg(y, src, dst2d):
    mesh = plsc.VectorSubcoreMesh(core_axis_name="c", subcore_axis_name="s")

    @functools.partial(
        pl.kernel,
        out_type=[
            jax.ShapeDtypeStruct((_NSC, _NPAD, _H), jnp.float32),
            jax.ShapeDtypeStruct((_NSC, _NPAD), jnp.float32),
        ],
        mesh=mesh,
        compiler_params=pltpu.CompilerParams(use_tc_tiling_on_sc=False),
        scratch_types=[
            pltpu.VMEM_SHARED((_NPAD, _H), jnp.float32),
            pltpu.VMEM_SHARED((_NPAD,), jnp.float32),
            pltpu.VMEM((_EPW,), jnp.int32),
            pltpu.VMEM((_NCH, _K), jnp.int32),
            pltpu.VMEM((_NBUF, _K, _H), jnp.float32),
            pltpu.VMEM((_K,), jnp.float32),
            pltpu.VMEM((_K,), jnp.float32),
            pltpu.SemaphoreType.DMA,
            pltpu.SemaphoreType.DMA((_NBUF,)),
            pltpu.SemaphoreType.DMA((_NBUF,)),
            pltpu.SemaphoreType.DMA,
        ],
    )
    def k(y_hbm, src_hbm, dst_hbm, acc_out, deg_out,
          acc_sh, deg_sh, srcv, dstv, rows, onev, zerov,
          semi, semg, sems, semd):
        cid = lax.axis_index("c")
        sid = lax.axis_index("s")
        wid = cid * _NTILE + sid

        cp_s = pltpu.async_copy(src_hbm.at[pl.ds(wid * _EPW, _EPW)], srcv, semi)
        cp_d = pltpu.async_copy(dst_hbm.at[pl.ds(wid * _NCH, _NCH)], dstv, semi)

        def fill_rows(i, _):
            rows[0, i] = jnp.zeros((_H,), jnp.float32)
            return 0

        lax.fori_loop(0, _K, fill_rows, 0)

        def fill_vecs(i, _):
            zerov[pl.ds(i * 16, 16)] = jnp.zeros((16,), jnp.float32)
            onev[pl.ds(i * 16, 16)] = jnp.ones((16,), jnp.float32)
            return 0

        lax.fori_loop(0, _K // 16, fill_vecs, 0)

        def zero_stripe(i, _):
            off = sid * _RPT + i * _K
            pltpu.sync_copy(rows.at[0], acc_sh.at[pl.ds(off, _K)])
            pltpu.sync_copy(zerov, deg_sh.at[pl.ds(off, _K)])
            return 0

        lax.fori_loop(0, _RPT // _K, zero_stripe, 0)
        cp_s.wait()
        cp_d.wait()
        plsc.subcore_barrier()

        def gather(j, b):
            sidx = srcv.at[pl.ds(j * _K, _K)]
            pltpu.async_copy(y_hbm.at[sidx], rows.at[b], semg.at[b])

        for b in range(_NBUF):
            gather(b, b)

        half = _NBUF // 2

        def outer(i, _):
            j0 = i * _NBUF
            for b in range(_NBUF):
                j = j0 + b
                sidx = srcv.at[pl.ds(j * _K, _K)]
                pltpu.make_async_copy(
                    y_hbm.at[sidx], rows.at[b], semg.at[b]).wait()
                dvi = dstv.at[j]
                pltpu.async_copy(rows.at[b], acc_sh.at[dvi], sems.at[b],
                                 add=True)
                pltpu.async_copy(onev, deg_sh.at[dvi], semd, add=True)

                @pl.when(j >= half)
                def _():
                    pltpu.make_async_copy(
                        onev, deg_sh.at[dvi], semd).wait()

                bp = (b + half) % _NBUF

                @pl.when((j >= half) & (j + half < _NCH))
                def _():
                    pltpu.make_async_copy(
                        rows.at[bp], acc_sh.at[dvi], sems.at[bp]).wait()
                    gather(j + half, bp)

            return 0

        lax.fori_loop(0, _NCH // _NBUF, outer, 0)

        for b in range(half):
            pltpu.make_async_copy(onev, deg_sh.at[dstv.at[0]], semd).wait()
        for b in range(_NBUF):
            pltpu.make_async_copy(
                rows.at[b], acc_sh.at[dstv.at[0]], sems.at[b]).wait()
        plsc.subcore_barrier()

        off = sid * _RPT
        pltpu.sync_copy(acc_sh.at[pl.ds(off, _RPT)],
                        acc_out.at[cid, pl.ds(off, _RPT)])
        pltpu.sync_copy(deg_sh.at[pl.ds(off, _RPT)],
                        deg_out.at[cid, pl.ds(off, _RPT)])

    return k(y, src, dst2d)


def _tc_post(acc, deg3, z, wfct, bfc):
    def body(acc_ref, deg_ref, z_ref, w_ref, b_ref, out_ref):
        a = acc_ref[0] + acc_ref[1]
        d = deg_ref[0] + deg_ref[1]
        h = jnp.maximum(a[:_N] / jnp.maximum(d[:_N], 1.0) + z_ref[...], 0.0)
        out_ref[...] = (
            jnp.dot(h, w_ref[...], preferred_element_type=jnp.float32)
            + b_ref[...]
        )

    return pl.pallas_call(
        body,
        out_shape=jax.ShapeDtypeStruct((_N, _C), jnp.float32),
    )(acc, deg3, z, wfct, bfc)


def kernel(x, edge_index, W_l, b_l, W_r, W_fc, b_fc):
    npad = _EPAD - _E
    it = lax.iota(jnp.int32, npad)
    src = jnp.concatenate([edge_index[0], it % _N])
    dst = jnp.concatenate([edge_index[1], _N + it % (_NPAD - _N)])
    dst2d = dst.reshape(_EPAD // _K, _K)
    y, z = _tc_pre(x, W_l.T, W_r.T, b_l.reshape(1, _H))
    acc, deg = _sc_agg(y, src, dst2d)
    return _tc_post(acc, deg.reshape(_NSC, _NPAD, 1), z, W_fc.T, b_fc.reshape(1, _C))


# speedup vs baseline: 1.5062x; 1.0960x over previous
import functools

import jax
import jax.numpy as jnp
from jax import lax
from jax.experimental import pallas as pl
from jax.experimental.pallas import tpu as pltpu
from jax.experimental.pallas import tpu_sc as plsc

_N, _F, _H, _C, _E = 10000, 128, 16, 16, 320000
_NPAD = 10240
_K = 128
_NSC, _NTILE = 2, 16
_NCHMAX = 79
_RPT = _NPAD // _NTILE
_NBUF = 8


def _tc_pre(x, wlt, wrt, bl):
    def body(x_ref, wl_ref, wr_ref, bl_ref, y_ref, z_ref):
        xv = x_ref[...]
        y_ref[...] = jnp.dot(xv, wl_ref[...], preferred_element_type=jnp.float32)
        z_ref[...] = (
            jnp.dot(xv, wr_ref[...], preferred_element_type=jnp.float32)
            + bl_ref[...]
        )

    return pl.pallas_call(
        body,
        out_shape=[
            jax.ShapeDtypeStruct((_N, _H), jnp.float32),
            jax.ShapeDtypeStruct((_N, _H), jnp.float32),
        ],
    )(x, wlt, wrt, bl)


def _sc_agg(y, edge_index):
    mesh = plsc.VectorSubcoreMesh(core_axis_name="c", subcore_axis_name="s")

    @functools.partial(
        pl.kernel,
        out_type=[
            jax.ShapeDtypeStruct((_NSC, _NPAD, _H), jnp.float32),
            jax.ShapeDtypeStruct((_NSC, _NPAD), jnp.float32),
        ],
        mesh=mesh,
        compiler_params=pltpu.CompilerParams(use_tc_tiling_on_sc=False),
        scratch_types=[
            pltpu.VMEM_SHARED((_NPAD, _H), jnp.float32),
            pltpu.VMEM_SHARED((_NPAD,), jnp.float32),
            pltpu.VMEM((_NCHMAX * _K,), jnp.int32),
            pltpu.VMEM((_NCHMAX * _K,), jnp.int32),
            pltpu.VMEM((_NBUF, _K, _H), jnp.float32),
            pltpu.VMEM((_K,), jnp.float32),
            pltpu.VMEM((_K,), jnp.float32),
            pltpu.SemaphoreType.DMA,
            pltpu.SemaphoreType.DMA((_NBUF,)),
            pltpu.SemaphoreType.DMA((_NBUF,)),
            pltpu.SemaphoreType.DMA,
        ],
    )
    def k(y_hbm, ei_hbm, acc_out, deg_out,
          acc_sh, deg_sh, srcv, dstv, rows, onev, zerov,
          semi, semg, sems, semd):
        cid = lax.axis_index("c")
        sid = lax.axis_index("s")
        wid = cid * _NTILE + sid

        nch = jnp.where(wid < 4, _NCHMAX, _NCHMAX - 1)
        base_chunk = jnp.where(
            wid < 4, _NCHMAX * wid, 4 * _NCHMAX + (_NCHMAX - 1) * (wid - 4))
        base = base_chunk * _K
        nmin = (_NCHMAX - 1) * _K

        cp_s = pltpu.async_copy(
            ei_hbm.at[0, pl.ds(base, nmin)], srcv.at[pl.ds(0, nmin)], semi)
        cp_d = pltpu.async_copy(
            ei_hbm.at[1, pl.ds(base, nmin)], dstv.at[pl.ds(0, nmin)], semi)

        def fill_rows(i, _):
            rows[0, i] = jnp.zeros((_H,), jnp.float32)
            return 0

        lax.fori_loop(0, _K, fill_rows, 0)

        def fill_vecs(i, _):
            zerov[pl.ds(i * 16, 16)] = jnp.zeros((16,), jnp.float32)
            onev[pl.ds(i * 16, 16)] = jnp.ones((16,), jnp.float32)
            return 0

        lax.fori_loop(0, _K // 16, fill_vecs, 0)

        def zero_stripe(i, _):
            off = sid * _RPT + i * _K
            pltpu.sync_copy(rows.at[0], acc_sh.at[pl.ds(off, _K)])
            pltpu.sync_copy(zerov, deg_sh.at[pl.ds(off, _K)])
            return 0

        lax.fori_loop(0, _RPT // _K, zero_stripe, 0)

        @pl.when(wid < 4)
        def _():
            pltpu.sync_copy(ei_hbm.at[0, pl.ds(base + nmin, _K)],
                            srcv.at[pl.ds(nmin, _K)])
            pltpu.sync_copy(ei_hbm.at[1, pl.ds(base + nmin, _K)],
                            dstv.at[pl.ds(nmin, _K)])

        cp_s.wait()
        cp_d.wait()
        plsc.subcore_barrier()

        half = _NBUF // 2

        def gather(j, b):
            sidx = srcv.at[pl.ds(j * _K, _K)]
            pltpu.async_copy(y_hbm.at[sidx], rows.at[b], semg.at[b])

        for b in range(half):
            gather(b, b)

        def step(j, _):
            b = lax.rem(j, _NBUF)
            bp = lax.rem(j + half, _NBUF)
            sidx = srcv.at[pl.ds(j * _K, _K)]
            pltpu.make_async_copy(y_hbm.at[sidx], rows.at[b], semg.at[b]).wait()
            dvi = dstv.at[pl.ds(j * _K, _K)]
            pltpu.async_copy(rows.at[b], acc_sh.at[dvi], sems.at[b], add=True)
            pltpu.async_copy(onev, deg_sh.at[dvi], semd, add=True)

            @pl.when(j >= half)
            def _():
                pltpu.make_async_copy(onev, deg_sh.at[dvi], semd).wait()

            @pl.when((j >= half) & (j + half < nch))
            def _():
                pltpu.make_async_copy(
                    rows.at[bp], acc_sh.at[dvi], sems.at[bp]).wait()

            @pl.when(j + half < nch)
            def _():
                gather(j + half, bp)

            return 0

        lax.fori_loop(0, nch, step, 0)

        for b in range(half):
            pltpu.make_async_copy(onev, deg_sh.at[dstv.at[pl.ds(0, _K)]], semd).wait()
        for b in range(_NBUF):
            pltpu.make_async_copy(
                rows.at[b], acc_sh.at[dstv.at[pl.ds(0, _K)]], sems.at[b]).wait()
        plsc.subcore_barrier()

        off = sid * _RPT
        pltpu.sync_copy(acc_sh.at[pl.ds(off, _RPT)],
                        acc_out.at[cid, pl.ds(off, _RPT)])
        pltpu.sync_copy(deg_sh.at[pl.ds(off, _RPT)],
                        deg_out.at[cid, pl.ds(off, _RPT)])

    return k(y, edge_index)


def _tc_post(acc, deg3, z, wfct, bfc):
    def body(acc_ref, deg_ref, z_ref, w_ref, b_ref, out_ref):
        a = acc_ref[0] + acc_ref[1]
        d = deg_ref[0] + deg_ref[1]
        h = jnp.maximum(a[:_N] / jnp.maximum(d[:_N], 1.0) + z_ref[...], 0.0)
        out_ref[...] = (
            jnp.dot(h, w_ref[...], preferred_element_type=jnp.float32)
            + b_ref[...]
        )

    return pl.pallas_call(
        body,
        out_shape=jax.ShapeDtypeStruct((_N, _C), jnp.float32),
    )(acc, deg3, z, wfct, bfc)


def kernel(x, edge_index, W_l, b_l, W_r, W_fc, b_fc):
    y, z = _tc_pre(x, W_l.T, W_r.T, b_l.reshape(1, _H))
    acc, deg = _sc_agg(y, edge_index)
    return _tc_post(acc, deg.reshape(_NSC, _NPAD, 1), z, W_fc.T, b_fc.reshape(1, _C))


# speedup vs baseline: 1.9690x; 1.3072x over previous
import functools

import jax
import jax.numpy as jnp
from jax import lax
from jax.experimental import pallas as pl
from jax.experimental.pallas import tpu as pltpu
from jax.experimental.pallas import tpu_sc as plsc

_N, _F, _H, _C, _E = 10000, 128, 16, 16, 320000
_NPAD = 10240
_K = 128
_NSC, _NTILE = 2, 16
_NCHMAX = 79
_RPT = _NPAD // _NTILE
_NBUF = 8
_LP = 128 // _H
_NP = _N // _LP
_NPP = _NPAD // _LP


def _tc_pre(x_r, w1, w2, bl_p):
    def body(x_ref, w1_ref, w2_ref, bl_ref, y_ref, z_ref):
        xv = x_ref[...]
        y_ref[...] = jnp.dot(xv, w1_ref[...], preferred_element_type=jnp.float32)
        z_ref[...] = (
            jnp.dot(xv, w2_ref[...], preferred_element_type=jnp.float32)
            + bl_ref[...]
        )

    return pl.pallas_call(
        body,
        out_shape=[
            jax.ShapeDtypeStruct((_NP, 128), jnp.float32),
            jax.ShapeDtypeStruct((_NP, 128), jnp.float32),
        ],
    )(x_r, w1, w2, bl_p)


def _sc_agg(y, edge_index):
    mesh = plsc.VectorSubcoreMesh(core_axis_name="c", subcore_axis_name="s")

    @functools.partial(
        pl.kernel,
        out_type=[
            jax.ShapeDtypeStruct((_NSC, _NPAD, _H), jnp.float32),
            jax.ShapeDtypeStruct((_NSC, _NPAD, _H), jnp.float32),
        ],
        mesh=mesh,
        compiler_params=pltpu.CompilerParams(use_tc_tiling_on_sc=False),
        scratch_types=[
            pltpu.VMEM_SHARED((_NPAD, _H), jnp.float32),
            pltpu.VMEM_SHARED((_NPAD, _H), jnp.float32),
            pltpu.VMEM((_NCHMAX * _K,), jnp.int32),
            pltpu.VMEM((_NCHMAX * _K,), jnp.int32),
            pltpu.VMEM((_NBUF, _K, _H), jnp.float32),
            pltpu.VMEM((_K, _H), jnp.float32),
            pltpu.SemaphoreType.DMA,
            pltpu.SemaphoreType.DMA((_NBUF,)),
            pltpu.SemaphoreType.DMA((_NBUF,)),
            pltpu.SemaphoreType.DMA,
        ],
    )
    def k(y_hbm, ei_hbm, acc_out, deg_out,
          acc_sh, deg_sh, srcv, dstv, rows, onev,
          semi, semg, sems, semd):
        cid = lax.axis_index("c")
        sid = lax.axis_index("s")
        wid = cid * _NTILE + sid

        nch = jnp.where(wid < 4, _NCHMAX, _NCHMAX - 1)
        base_chunk = jnp.where(
            wid < 4, _NCHMAX * wid, 4 * _NCHMAX + (_NCHMAX - 1) * (wid - 4))
        base = base_chunk * _K
        nmin = (_NCHMAX - 1) * _K

        cp_s = pltpu.async_copy(
            ei_hbm.at[0, pl.ds(base, nmin)], srcv.at[pl.ds(0, nmin)], semi)
        cp_d = pltpu.async_copy(
            ei_hbm.at[1, pl.ds(base, nmin)], dstv.at[pl.ds(0, nmin)], semi)

        def fill_rows(i, _):
            rows[0, i] = jnp.zeros((_H,), jnp.float32)
            onev[i] = jnp.ones((_H,), jnp.float32)
            return 0

        lax.fori_loop(0, _K, fill_rows, 0)

        def zero_stripe(i, _):
            off = sid * _RPT + i * _K
            pltpu.sync_copy(rows.at[0], acc_sh.at[pl.ds(off, _K)])
            pltpu.sync_copy(rows.at[0], deg_sh.at[pl.ds(off, _K)])
            return 0

        lax.fori_loop(0, _RPT // _K, zero_stripe, 0)

        @pl.when(wid < 4)
        def _():
            pltpu.sync_copy(ei_hbm.at[0, pl.ds(base + nmin, _K)],
                            srcv.at[pl.ds(nmin, _K)])
            pltpu.sync_copy(ei_hbm.at[1, pl.ds(base + nmin, _K)],
                            dstv.at[pl.ds(nmin, _K)])

        cp_s.wait()
        cp_d.wait()
        plsc.subcore_barrier()

        half = _NBUF // 2

        def gather(j, b):
            sidx = srcv.at[pl.ds(j * _K, _K)]
            pltpu.async_copy(y_hbm.at[sidx], rows.at[b], semg.at[b])

        for b in range(half):
            gather(b, b)

        def step(j, _):
            b = lax.rem(j, _NBUF)
            bp = lax.rem(j + half, _NBUF)
            sidx = srcv.at[pl.ds(j * _K, _K)]
            pltpu.make_async_copy(y_hbm.at[sidx], rows.at[b], semg.at[b]).wait()
            dvi = dstv.at[pl.ds(j * _K, _K)]
            pltpu.async_copy(rows.at[b], acc_sh.at[dvi], sems.at[b], add=True)
            pltpu.async_copy(onev, deg_sh.at[dvi], semd, add=True)

            @pl.when(j >= half)
            def _():
                pltpu.make_async_copy(onev, deg_sh.at[dvi], semd).wait()

            @pl.when((j >= half) & (j + half < nch))
            def _():
                pltpu.make_async_copy(
                    rows.at[bp], acc_sh.at[dvi], sems.at[bp]).wait()

            @pl.when(j + half < nch)
            def _():
                gather(j + half, bp)

            return 0

        lax.fori_loop(0, nch, step, 0)

        dv0 = dstv.at[pl.ds(0, _K)]
        for b in range(half):
            pltpu.make_async_copy(onev, deg_sh.at[dv0], semd).wait()
        for b in range(_NBUF):
            pltpu.make_async_copy(rows.at[b], acc_sh.at[dv0], sems.at[b]).wait()
        plsc.subcore_barrier()

        off = sid * _RPT
        pltpu.sync_copy(acc_sh.at[pl.ds(off, _RPT)],
                        acc_out.at[cid, pl.ds(off, _RPT)])
        pltpu.sync_copy(deg_sh.at[pl.ds(off, _RPT)],
                        deg_out.at[cid, pl.ds(off, _RPT)])

    return k(y, edge_index)


def _tc_post(acc, deg, z, w3, bfc_p):
    def body(acc_ref, deg_ref, z_ref, w_ref, b_ref, out_ref):
        a = acc_ref[0] + acc_ref[1]
        d = deg_ref[0] + deg_ref[1]
        h = jnp.maximum(a[:_NP] / jnp.maximum(d[:_NP], 1.0) + z_ref[...], 0.0)
        out_ref[...] = (
            jnp.dot(h, w_ref[...], preferred_element_type=jnp.float32)
            + b_ref[...]
        )

    return pl.pallas_call(
        body,
        out_shape=jax.ShapeDtypeStruct((_NP, 128), jnp.float32),
    )(acc, deg, z, w3, bfc_p)


def kernel(x, edge_index, W_l, b_l, W_r, W_fc, b_fc):
    eye = jnp.eye(_LP, dtype=jnp.float32)
    w1 = jnp.kron(eye, W_l.T)
    w2 = jnp.kron(eye, W_r.T)
    w3 = jnp.kron(eye, W_fc.T)
    bl_p = jnp.tile(b_l, _LP).reshape(1, 128)
    bfc_p = jnp.tile(b_fc, _LP).reshape(1, 128)
    x_r = x.reshape(_NP, _LP * _F)
    y_p, z_p = _tc_pre(x_r, w1, w2, bl_p)
    acc, deg = _sc_agg(y_p.reshape(_N, _H), edge_index)
    out_p = _tc_post(acc.reshape(_NSC, _NPP, 128),
                     deg.reshape(_NSC, _NPP, 128), z_p, w3, bfc_p)
    return out_p.reshape(_N, _C)


# speedup vs baseline: 2.0242x; 1.0280x over previous
import functools

import jax
import jax.numpy as jnp
from jax import lax
from jax.experimental import pallas as pl
from jax.experimental.pallas import tpu as pltpu
from jax.experimental.pallas import tpu_sc as plsc

_N, _F, _H, _C, _E = 10000, 128, 16, 16, 320000
_NPAD = 10240
_K = 128
_NSC, _NTILE = 2, 16
_NCHMAX = 79
_RPT = _NPAD // _NTILE
_NBUF = 12
_LP = 128 // _H
_NP = _N // _LP
_NPP = _NPAD // _LP


def _tc_pre(x_r, w1, w2, bl_p):
    def body(x_ref, w1_ref, w2_ref, bl_ref, y_ref, z_ref):
        xv = x_ref[...]
        y_ref[...] = jnp.dot(xv, w1_ref[...], preferred_element_type=jnp.float32)
        z_ref[...] = (
            jnp.dot(xv, w2_ref[...], preferred_element_type=jnp.float32)
            + bl_ref[...]
        )

    return pl.pallas_call(
        body,
        out_shape=[
            jax.ShapeDtypeStruct((_NP, 128), jnp.float32),
            jax.ShapeDtypeStruct((_NP, 128), jnp.float32),
        ],
    )(x_r, w1, w2, bl_p)


def _sc_agg(y, edge_index):
    mesh = plsc.VectorSubcoreMesh(core_axis_name="c", subcore_axis_name="s")

    @functools.partial(
        pl.kernel,
        out_type=[
            jax.ShapeDtypeStruct((_NSC, _NPAD, _H), jnp.float32),
            jax.ShapeDtypeStruct((_NSC, _NPAD, _H), jnp.float32),
        ],
        mesh=mesh,
        compiler_params=pltpu.CompilerParams(use_tc_tiling_on_sc=False),
        scratch_types=[
            pltpu.VMEM_SHARED((_NPAD, _H), jnp.float32),
            pltpu.VMEM_SHARED((_NPAD, _H), jnp.float32),
            pltpu.VMEM((_NCHMAX * _K,), jnp.int32),
            pltpu.VMEM((_NCHMAX * _K,), jnp.int32),
            pltpu.VMEM((_NBUF, _K, _H), jnp.float32),
            pltpu.VMEM((_K, _H), jnp.float32),
            pltpu.SemaphoreType.DMA,
            pltpu.SemaphoreType.DMA((_NBUF,)),
            pltpu.SemaphoreType.DMA((_NBUF,)),
            pltpu.SemaphoreType.DMA,
        ],
    )
    def k(y_hbm, ei_hbm, acc_out, deg_out,
          acc_sh, deg_sh, srcv, dstv, rows, onev,
          semi, semg, sems, semd):
        cid = lax.axis_index("c")
        sid = lax.axis_index("s")
        wid = cid * _NTILE + sid

        nch = jnp.where(wid < 4, _NCHMAX, _NCHMAX - 1)
        base_chunk = jnp.where(
            wid < 4, _NCHMAX * wid, 4 * _NCHMAX + (_NCHMAX - 1) * (wid - 4))
        base = base_chunk * _K
        nmin = (_NCHMAX - 1) * _K

        cp_s = pltpu.async_copy(
            ei_hbm.at[0, pl.ds(base, nmin)], srcv.at[pl.ds(0, nmin)], semi)
        cp_d = pltpu.async_copy(
            ei_hbm.at[1, pl.ds(base, nmin)], dstv.at[pl.ds(0, nmin)], semi)

        def fill_rows(i, _):
            rows[0, i] = jnp.zeros((_H,), jnp.float32)
            onev[i] = jnp.ones((_H,), jnp.float32)
            return 0

        lax.fori_loop(0, _K, fill_rows, 0)

        def zero_stripe(i, _):
            off = sid * _RPT + i * _K
            pltpu.sync_copy(rows.at[0], acc_sh.at[pl.ds(off, _K)])
            pltpu.sync_copy(rows.at[0], deg_sh.at[pl.ds(off, _K)])
            return 0

        lax.fori_loop(0, _RPT // _K, zero_stripe, 0)

        @pl.when(wid < 4)
        def _():
            pltpu.sync_copy(ei_hbm.at[0, pl.ds(base + nmin, _K)],
                            srcv.at[pl.ds(nmin, _K)])
            pltpu.sync_copy(ei_hbm.at[1, pl.ds(base + nmin, _K)],
                            dstv.at[pl.ds(nmin, _K)])

        cp_s.wait()
        cp_d.wait()
        plsc.subcore_barrier()

        half = _NBUF // 2

        def gather(j, b):
            sidx = srcv.at[pl.ds(j * _K, _K)]
            pltpu.async_copy(y_hbm.at[sidx], rows.at[b], semg.at[b])

        for b in range(half):
            gather(b, b)

        def step(j, _):
            b = lax.rem(j, _NBUF)
            bp = lax.rem(j + half, _NBUF)
            sidx = srcv.at[pl.ds(j * _K, _K)]
            pltpu.make_async_copy(y_hbm.at[sidx], rows.at[b], semg.at[b]).wait()
            dvi = dstv.at[pl.ds(j * _K, _K)]
            pltpu.async_copy(rows.at[b], acc_sh.at[dvi], sems.at[b], add=True)
            pltpu.async_copy(onev, deg_sh.at[dvi], semd, add=True)

            @pl.when(j >= half)
            def _():
                pltpu.make_async_copy(onev, deg_sh.at[dvi], semd).wait()

            @pl.when((j >= half) & (j + half < nch))
            def _():
                pltpu.make_async_copy(
                    rows.at[bp], acc_sh.at[dvi], sems.at[bp]).wait()

            @pl.when(j + half < nch)
            def _():
                gather(j + half, bp)

            return 0

        lax.fori_loop(0, nch, step, 0)

        dv0 = dstv.at[pl.ds(0, _K)]
        for b in range(half):
            pltpu.make_async_copy(onev, deg_sh.at[dv0], semd).wait()
        for b in range(_NBUF):
            pltpu.make_async_copy(rows.at[b], acc_sh.at[dv0], sems.at[b]).wait()
        plsc.subcore_barrier()

        off = sid * _RPT
        pltpu.sync_copy(acc_sh.at[pl.ds(off, _RPT)],
                        acc_out.at[cid, pl.ds(off, _RPT)])
        pltpu.sync_copy(deg_sh.at[pl.ds(off, _RPT)],
                        deg_out.at[cid, pl.ds(off, _RPT)])

    return k(y, edge_index)


def _tc_post(acc, deg, z, w3, bfc_p):
    def body(acc_ref, deg_ref, z_ref, w_ref, b_ref, out_ref):
        a = acc_ref[0] + acc_ref[1]
        d = deg_ref[0] + deg_ref[1]
        h = jnp.maximum(a[:_NP] / jnp.maximum(d[:_NP], 1.0) + z_ref[...], 0.0)
        out_ref[...] = (
            jnp.dot(h, w_ref[...], preferred_element_type=jnp.float32)
            + b_ref[...]
        )

    return pl.pallas_call(
        body,
        out_shape=jax.ShapeDtypeStruct((_NP, 128), jnp.float32),
    )(acc, deg, z, w3, bfc_p)


def kernel(x, edge_index, W_l, b_l, W_r, W_fc, b_fc):
    eye = jnp.eye(_LP, dtype=jnp.float32)
    w1 = jnp.kron(eye, W_l.T)
    w2 = jnp.kron(eye, W_r.T)
    w3 = jnp.kron(eye, W_fc.T)
    bl_p = jnp.tile(b_l, _LP).reshape(1, 128)
    bfc_p = jnp.tile(b_fc, _LP).reshape(1, 128)
    x_r = x.reshape(_NP, _LP * _F)
    y_p, z_p = _tc_pre(x_r, w1, w2, bl_p)
    acc, deg = _sc_agg(y_p.reshape(_N, _H), edge_index)
    out_p = _tc_post(acc.reshape(_NSC, _NPP, 128),
                     deg.reshape(_NSC, _NPP, 128), z_p, w3, bfc_p)
    return out_p.reshape(_N, _C)
